# baseline (speedup 1.0000x reference)
import functools

import jax
import jax.numpy as jnp
from jax import lax
from jax.experimental import pallas as pl
from jax.experimental.pallas import tpu as pltpu
from jax.experimental.pallas import tpu_sc as plsc

_B = 16384
_D = 16
_V = 100000

_info = plsc.get_sparse_core_info()
_NC = _info.num_cores
_NS = _info.num_subcores
_NW = _NC * _NS
_BPW = _B // _NW
_NT = _BPW // 128

_mesh = plsc.VectorSubcoreMesh(core_axis_name="c", subcore_axis_name="s")


@functools.partial(
    pl.kernel,
    mesh=_mesh,
    compiler_params=pltpu.CompilerParams(use_tc_tiling_on_sc=False),
    out_type=jax.ShapeDtypeStruct((2, _B, 8), jnp.float32),
    scratch_types=[
        pltpu.VMEM((_BPW,), jnp.int32),
        pltpu.VMEM((2, _NT, 128), jnp.int32),
        pltpu.VMEM((_BPW, 8), jnp.float32),
        pltpu.VMEM((_BPW, 8), jnp.float32),
        pltpu.SemaphoreType.DMA,
    ],
)
def _gather_one(tab, ids, out, idv, idxb, rbuf0, rbuf1, sem):
    wid = lax.axis_index("s") * _NC + lax.axis_index("c")
    base = wid * _BPW
    pltpu.sync_copy(ids.at[pl.ds(base, _BPW)], idv)
    for o in range(_NT):
        for s in range(8):
            vec = idv[pl.ds(o * 128 + s * 16, 16)]
            idxb[0, o, pl.ds(s * 16, 16)] = vec
            idxb[1, o, pl.ds(s * 16, 16)] = vec + _V
    copies = []
    for o in range(_NT):
        copies.append(pltpu.async_copy(
            tab.at[idxb.at[0, o]], rbuf0.at[pl.ds(o * 128, 128)], sem))
        copies.append(pltpu.async_copy(
            tab.at[idxb.at[1, o]], rbuf1.at[pl.ds(o * 128, 128)], sem))
    for cp in copies:
        cp.wait()
    pltpu.sync_copy(rbuf0, out.at[0, pl.ds(base, _BPW)])
    pltpu.sync_copy(rbuf1, out.at[1, pl.ds(base, _BPW)])


def _halfrows(table):
    return table.T.reshape(2, 8, _V).transpose(0, 2, 1).reshape(2 * _V, 8)


def kernel(contributor_table, recipient_table, contributor_ids, recipient_ids):
    r8 = _halfrows(recipient_table)
    c8 = _halfrows(contributor_table)
    xr3 = _gather_one(r8, recipient_ids.astype(jnp.int32))
    xc3 = _gather_one(c8, contributor_ids.astype(jnp.int32))
    xr = xr3.transpose(1, 0, 2).reshape(_B, _D)
    xc = xc3.transpose(1, 0, 2).reshape(_B, _D)
    return (xr, xc)


# speedup vs baseline: 3.8687x; 3.8687x over previous
import functools

import jax
import jax.numpy as jnp
from jax import lax
from jax.experimental import pallas as pl
from jax.experimental.pallas import tpu as pltpu
from jax.experimental.pallas import tpu_sc as plsc

_B = 16384
_D = 16
_V = 100000

_info = plsc.get_sparse_core_info()
_NC = _info.num_cores
_NS = _info.num_subcores
_NW = _NC * _NS
_BPW = _B // _NW
_NT = _BPW // 128

_mesh = plsc.VectorSubcoreMesh(core_axis_name="c", subcore_axis_name="s")


@functools.partial(
    pl.kernel,
    mesh=_mesh,
    compiler_params=pltpu.CompilerParams(use_tc_tiling_on_sc=True),
    out_type=jax.ShapeDtypeStruct((_D, _B), jnp.float32),
    scratch_types=[
        pltpu.VMEM((_BPW,), jnp.int32),
        pltpu.VMEM((_NT, 8, 128), jnp.int32),
        pltpu.VMEM((_NT, 8, 128), jnp.int32),
        pltpu.VMEM((8, _BPW), jnp.float32),
        pltpu.VMEM((8, _BPW), jnp.float32),
        pltpu.SemaphoreType.DMA,
        pltpu.SemaphoreType.DMA,
    ],
)
def _gather_one(tab, ids, out, idv, idxb0, idxb1, rbuf0, rbuf1, sem0, sem1):
    wid = lax.axis_index("s") * _NC + lax.axis_index("c")
    base = wid * _BPW
    pltpu.sync_copy(ids.at[pl.ds(base, _BPW)], idv)
    copies0, copies1 = [], []
    for o in range(_NT):
        for s in range(8):
            vec = idv[pl.ds(o * 128 + s * 16, 16)]
            for dd in range(8):
                idxb0[o, dd, pl.ds(s * 16, 16)] = vec + (dd * _V)
                idxb1[o, dd, pl.ds(s * 16, 16)] = vec + ((8 + dd) * _V)
        for dd in range(8):
            copies0.append(pltpu.async_copy(
                tab.at[idxb0.at[o, dd]],
                rbuf0.at[dd, pl.ds(o * 128, 128)], sem0))
            copies1.append(pltpu.async_copy(
                tab.at[idxb1.at[o, dd]],
                rbuf1.at[dd, pl.ds(o * 128, 128)], sem1))
    for cp in copies0:
        cp.wait()
    pltpu.sync_copy(rbuf0, out.at[pl.ds(0, 8), pl.ds(base, _BPW)])
    for cp in copies1:
        cp.wait()
    pltpu.sync_copy(rbuf1, out.at[pl.ds(8, 8), pl.ds(base, _BPW)])


def kernel(contributor_table, recipient_table, contributor_ids, recipient_ids):
    rflat = recipient_table.T.reshape(-1)
    cflat = contributor_table.T.reshape(-1)
    xrT = _gather_one(rflat, recipient_ids.astype(jnp.int32))
    xcT = _gather_one(cflat, contributor_ids.astype(jnp.int32))
    return (xrT.T, xcT.T)
